# baseline (speedup 1.0000x reference)
import jax
import jax.numpy as jnp
from jax import lax
from jax.experimental import pallas as pl
from jax.experimental.pallas import tpu as pltpu
from jax.experimental.pallas import tpu_sc as plsc

BATCH = 4096
SEQLEN = 200
EMBED = 64

NC = 2
NS = 16
NW = NC * NS
ROWS_PER_W = BATCH // NW
RB = 2
ITERS = ROWS_PER_W // (2 * RB)
TAIL_L0 = SEQLEN - 16


def _sc_body(x_hbm, pos_hbm, out_hbm, x_v, pos_v, buf0, buf1, sem0, sem1):
    wid = lax.axis_index("s") * NC + lax.axis_index("c")
    base = wid * ROWS_PER_W

    pltpu.sync_copy(pos_hbm, pos_v)
    pltpu.sync_copy(x_hbm.at[pl.ds(base, ROWS_PER_W)], x_v)

    def emit16(buf, rr, l0, xv):
        for j in range(16):
            l = l0 + j
            s = xv[j]
            for dd in range(4):
                sl = pl.ds(dd * 16, 16)
                buf[rr, l, sl] = pos_v[l, sl] + s

    def compute_pair(buf, r0):
        def chunk(lc, carry):
            l0 = pl.multiple_of(lc * 16, 16)
            for rr in range(RB):
                xv = x_v[r0 + rr, pl.ds(l0, 16)]
                emit16(buf, rr, l0, xv)
            return carry

        lax.fori_loop(0, SEQLEN // 16, chunk, 0)
        for rr in range(RB):
            xv = x_v[r0 + rr, pl.ds(TAIL_L0, 16)]
            emit16(buf, rr, TAIL_L0, xv)

    def body(i, carry):
        r0 = 2 * RB * i

        @pl.when(i > 0)
        def _():
            pltpu.make_async_copy(buf0, out_hbm.at[pl.ds(0, RB)], sem0).wait()

        compute_pair(buf0, r0)
        pltpu.make_async_copy(buf0, out_hbm.at[pl.ds(base + r0, RB)], sem0).start()

        @pl.when(i > 0)
        def _():
            pltpu.make_async_copy(buf1, out_hbm.at[pl.ds(0, RB)], sem1).wait()

        compute_pair(buf1, r0 + RB)
        pltpu.make_async_copy(buf1, out_hbm.at[pl.ds(base + r0 + RB, RB)], sem1).start()
        return carry

    lax.fori_loop(0, ITERS, body, 0)
    pltpu.make_async_copy(buf0, out_hbm.at[pl.ds(0, RB)], sem0).wait()
    pltpu.make_async_copy(buf1, out_hbm.at[pl.ds(0, RB)], sem1).wait()


def kernel(x, pos_table):
    mesh = plsc.VectorSubcoreMesh(core_axis_name="c", subcore_axis_name="s")
    k = pl.kernel(
        _sc_body,
        mesh=mesh,
        compiler_params=pltpu.CompilerParams(use_tc_tiling_on_sc=False),
        out_type=jax.ShapeDtypeStruct((BATCH, SEQLEN, EMBED), jnp.float32),
        scratch_types=[
            pltpu.VMEM((ROWS_PER_W, SEQLEN), jnp.float32),
            pltpu.VMEM((SEQLEN, EMBED), jnp.float32),
            pltpu.VMEM((RB, SEQLEN, EMBED), jnp.float32),
            pltpu.VMEM((RB, SEQLEN, EMBED), jnp.float32),
            pltpu.SemaphoreType.DMA,
            pltpu.SemaphoreType.DMA,
        ],
    )
    return k(x, pos_table)
